# baseline (speedup 1.0000x reference)
import functools
import math

import jax
import jax.numpy as jnp
from jax import lax
from jax.experimental import pallas as pl
from jax.experimental.pallas import tpu as pltpu
from jax.experimental.pallas import tpu_sc as plsc

VOCAB = 1000000
D_MODEL = 64
BATCH = 4096
HIST = 200

NC = 2
NS = 16
NW = NC * NS
B = BATCH * HIST
C = 128
RPW = B // NW
NCH = RPW // C
NSLOT = 4
SCALE = math.sqrt(float(D_MODEL))

_mesh = plsc.VectorSubcoreMesh(core_axis_name="c", subcore_axis_name="s")


@functools.partial(
    pl.kernel,
    out_type=jax.ShapeDtypeStruct((B, D_MODEL), jnp.float32),
    mesh=_mesh,
    compiler_params=pltpu.CompilerParams(use_tc_tiling_on_sc=False),
    scratch_types=[
        pltpu.VMEM((NSLOT, C), jnp.int32),
        pltpu.VMEM((NSLOT, C, D_MODEL), jnp.float32),
        pltpu.SemaphoreType.DMA,
        pltpu.SemaphoreType.DMA,
        pltpu.SemaphoreType.DMA,
        pltpu.SemaphoreType.DMA,
    ],
)
def _emb_kernel(lut_hbm, idx_hbm, out_hbm, idx_v, rows_v, s0, s1, s2, s3):
    sems = (s0, s1, s2, s3)
    wid = lax.axis_index("s") * NC + lax.axis_index("c")
    chunk0 = wid * NCH

    def fire(g, p):
        pltpu.sync_copy(idx_hbm.at[chunk0 + g], idx_v.at[p])
        pltpu.async_copy(lut_hbm.at[idx_v.at[p]], rows_v.at[p], sems[p])

    def wait_gather(p):
        pltpu.make_async_copy(lut_hbm.at[idx_v.at[p]], rows_v.at[p], sems[p]).wait()

    def scale_slot(p):
        def body(r, _):
            for j in range(D_MODEL // 16):
                ds = pl.ds(j * 16, 16)
                rows_v[p, r, ds] = rows_v[p, r, ds] * SCALE
            return 0
        lax.fori_loop(0, C, body, 0)

    for p in range(NSLOT - 1):
        fire(p, p)

    def step(t, _):
        for p in range(NSLOT):
            g = t * NSLOT + p
            gf = g + (NSLOT - 1)

            @pl.when(gf < NCH)
            def _():
                fire(gf, (p + NSLOT - 1) % NSLOT)

            wait_gather(p)
            scale_slot(p)
            pltpu.sync_copy(rows_v.at[p], out_hbm.at[pl.ds((chunk0 + g) * C, C)])
        return 0

    lax.fori_loop(0, NCH // NSLOT, step, 0)


def kernel(x, lut):
    idx2d = x.astype(jnp.int32).reshape(B // C, C)
    out = _emb_kernel(lut, idx2d)
    return out.reshape(BATCH, HIST, D_MODEL)


# speedup vs baseline: 1.1221x; 1.1221x over previous
import functools
import math

import jax
import jax.numpy as jnp
from jax import lax
from jax.experimental import pallas as pl
from jax.experimental.pallas import tpu as pltpu
from jax.experimental.pallas import tpu_sc as plsc

VOCAB = 1000000
D_MODEL = 64
BATCH = 4096
HIST = 200

NC = 2
NS = 16
NW = NC * NS
B = BATCH * HIST
C = 128
G4 = 4
R = C * G4
RPW = B // NW
NCH = RPW // C
NG = RPW // R
NBUF = 3
SCALE = math.sqrt(float(D_MODEL))

_mesh = plsc.VectorSubcoreMesh(core_axis_name="c", subcore_axis_name="s")


@functools.partial(
    pl.kernel,
    out_type=jax.ShapeDtypeStruct((B, D_MODEL), jnp.float32),
    mesh=_mesh,
    compiler_params=pltpu.CompilerParams(use_tc_tiling_on_sc=False),
    scratch_types=[
        pltpu.VMEM((NCH, C), jnp.int32),
        pltpu.VMEM((NBUF, R, D_MODEL), jnp.float32),
        pltpu.SemaphoreType.DMA,
        pltpu.SemaphoreType.DMA,
        pltpu.SemaphoreType.DMA,
        pltpu.SemaphoreType.DMA,
        pltpu.SemaphoreType.DMA,
        pltpu.SemaphoreType.DMA,
    ],
)
def _emb_kernel(lut_hbm, idx_hbm, out_hbm, idx_v, rows_v, g0, g1, g2, s0, s1, s2):
    gsem = (g0, g1, g2)
    ssem = (s0, s1, s2)
    wid = lax.axis_index("s") * NC + lax.axis_index("c")
    row0 = wid * RPW

    pltpu.sync_copy(idx_hbm.at[pl.ds(wid * NCH, NCH)], idx_v)

    def fire_group(g, b):
        for j in range(G4):
            pltpu.async_copy(
                lut_hbm.at[idx_v.at[g * G4 + j]],
                rows_v.at[b, pl.ds(j * C, C)],
                gsem[b],
            )

    def wait_group(g, b):
        for j in range(G4):
            pltpu.make_async_copy(
                lut_hbm.at[idx_v.at[g * G4 + j]],
                rows_v.at[b, pl.ds(j * C, C)],
                gsem[b],
            ).wait()

    def fire_scatter(g, b):
        pltpu.async_copy(rows_v.at[b], out_hbm.at[pl.ds(row0 + g * R, R)], ssem[b])

    def wait_scatter(g, b):
        pltpu.make_async_copy(rows_v.at[b], out_hbm.at[pl.ds(row0 + g * R, R)], ssem[b]).wait()

    def scale_buf(b):
        def body(i, _):
            for k in range(4):
                r = i * 4 + k
                for j in range(D_MODEL // 16):
                    ds = pl.ds(j * 16, 16)
                    rows_v[b, r, ds] = rows_v[b, r, ds] * SCALE
            return 0
        lax.fori_loop(0, R // 4, body, 0)

    fire_group(0, 0)
    fire_group(1, 1)

    def step(t, _):
        for b in range(NBUF):
            g = t * NBUF + b

            @pl.when(g < NG)
            def _():
                wait_group(g, b)
                scale_buf(b)
                fire_scatter(g, b)
                bn = (b + 2) % NBUF

                @pl.when(jnp.logical_and(g >= 1, g + 2 < NG))
                def _():
                    wait_scatter(g - 1, bn)

                @pl.when(g + 2 < NG)
                def _():
                    fire_group(g + 2, bn)
        return 0

    lax.fori_loop(0, (NG + NBUF - 1) // NBUF, step, 0)

    for g in (NG - 3, NG - 2, NG - 1):
        wait_scatter(g, g % NBUF)


def kernel(x, lut):
    idx2d = x.astype(jnp.int32).reshape(B // C, C)
    out = _emb_kernel(lut, idx2d)
    return out.reshape(BATCH, HIST, D_MODEL)


# speedup vs baseline: 1.4915x; 1.3292x over previous
import functools
import math

import jax
import jax.numpy as jnp
from jax import lax
from jax.experimental import pallas as pl
from jax.experimental.pallas import tpu as pltpu
from jax.experimental.pallas import tpu_sc as plsc

VOCAB = 1000000
D_MODEL = 64
BATCH = 4096
HIST = 200

NC = 2
NS = 16
NW = NC * NS
B = BATCH * HIST
C = 128
G4 = 4
R = C * G4
RPW = B // NW
NCH = RPW // C
NG = RPW // R
NBUF = 3
SCALE = math.sqrt(float(D_MODEL))

_mesh = plsc.VectorSubcoreMesh(core_axis_name="c", subcore_axis_name="s")


@functools.partial(
    pl.kernel,
    out_type=jax.ShapeDtypeStruct((B, 2 * D_MODEL), jnp.float32),
    mesh=_mesh,
    compiler_params=pltpu.CompilerParams(use_tc_tiling_on_sc=False),
    scratch_types=[
        pltpu.VMEM((NCH, C), jnp.int32),
        pltpu.VMEM((NBUF, R, D_MODEL), jnp.float32),
        pltpu.SemaphoreType.DMA,
        pltpu.SemaphoreType.DMA,
        pltpu.SemaphoreType.DMA,
        pltpu.SemaphoreType.DMA,
        pltpu.SemaphoreType.DMA,
        pltpu.SemaphoreType.DMA,
    ],
)
def _emb_kernel(lut_hbm, idx_hbm, out_hbm, idx_v, rows_v, g0, g1, g2, s0, s1, s2):
    gsem = (g0, g1, g2)
    ssem = (s0, s1, s2)
    wid = lax.axis_index("s") * NC + lax.axis_index("c")
    row0 = wid * RPW

    pltpu.sync_copy(idx_hbm.at[pl.ds(wid * NCH, NCH)], idx_v)

    def fire_group(g, b):
        for j in range(G4):
            pltpu.async_copy(
                lut_hbm.at[idx_v.at[g * G4 + j]],
                rows_v.at[b, pl.ds(j * C, C)],
                gsem[b],
            )

    def wait_group(g, b):
        for j in range(G4):
            pltpu.make_async_copy(
                lut_hbm.at[idx_v.at[g * G4 + j]],
                rows_v.at[b, pl.ds(j * C, C)],
                gsem[b],
            ).wait()

    def fire_scatter(g, b):
        pltpu.async_copy(rows_v.at[b], out_hbm.at[pl.ds(row0 + g * R, R), pl.ds(0, D_MODEL)], ssem[b])

    def wait_scatter(g, b):
        pltpu.make_async_copy(rows_v.at[b], out_hbm.at[pl.ds(row0 + g * R, R), pl.ds(0, D_MODEL)], ssem[b]).wait()

    def scale_buf(b):
        def body(i, _):
            for k in range(4):
                r = i * 4 + k
                for j in range(D_MODEL // 16):
                    ds = pl.ds(j * 16, 16)
                    rows_v[b, r, ds] = rows_v[b, r, ds] * SCALE
            return 0
        lax.fori_loop(0, R // 4, body, 0)

    fire_group(0, 0)
    fire_group(1, 1)

    def step(t, _):
        for b in range(NBUF):
            g = t * NBUF + b

            @pl.when(g < NG)
            def _():
                wait_group(g, b)
                scale_buf(b)
                fire_scatter(g, b)
                bn = (b + 2) % NBUF

                @pl.when(jnp.logical_and(g >= 1, g + 2 < NG))
                def _():
                    wait_scatter(g - 1, bn)

                @pl.when(g + 2 < NG)
                def _():
                    fire_group(g + 2, bn)
        return 0

    lax.fori_loop(0, (NG + NBUF - 1) // NBUF, step, 0)

    for g in (NG - 3, NG - 2, NG - 1):
        wait_scatter(g, g % NBUF)


def kernel(x, lut):
    idx2d = x.astype(jnp.int32).reshape(B // C, C)
    out = _emb_kernel(lut, idx2d)
    return out[:, :D_MODEL].reshape(BATCH, HIST, D_MODEL)
